# Initial kernel scaffold; baseline (speedup 1.0000x reference)
import jax
import jax.numpy as jnp
from jax.experimental import pallas as pl


def kernel(atoms, bonds, Ws, bs):
    raise NotImplementedError("write your pallas kernel here")


# speedup vs baseline: 48.7776x; 48.7776x over previous
import functools

import jax
import jax.numpy as jnp
from jax import lax
from jax.experimental import pallas as pl


def _tc_body(atoms_ref, bonds_ref, wp_ref, b_ref, out_ref, *, A, D, C):
    at = atoms_ref[0]
    bb = bonds_ref[0]
    col = lax.broadcasted_iota(jnp.int32, (A, A), 1)
    row = lax.broadcasted_iota(jnp.int32, (A, A), 0)
    M = (col == row).astype(jnp.float32)
    for d in range(D):
        M = M + (bb[:, d:d + 1] == col).astype(jnp.float32)
    summed = lax.dot_general(M, at, (((1,), (0,)), ((), ())),
                             preferred_element_type=jnp.float32)
    full = lax.dot_general(summed, wp_ref[...], (((1,), (0,)), ((), ())),
                           preferred_element_type=jnp.float32)
    full = full + b_ref[...]
    deg = jnp.sum((bb != -1).astype(jnp.int32), axis=1, keepdims=True)
    acc = jnp.zeros((A, C), jnp.float32)
    for d in range(D):
        acc = acc + jnp.where(deg == d, full[:, d * C:(d + 1) * C], 0.0)
    out_ref[0] = acc


def kernel(atoms, bonds, Ws, bs):
    S, A, F = atoms.shape
    D, _, C = Ws.shape
    wp = jnp.transpose(Ws, (1, 0, 2)).reshape(F, D * C)
    br = bs.reshape(1, D * C)
    bonds32 = bonds.astype(jnp.int32)
    return pl.pallas_call(
        functools.partial(_tc_body, A=A, D=D, C=C),
        grid=(S,),
        in_specs=[
            pl.BlockSpec((1, A, F), lambda s: (s, 0, 0)),
            pl.BlockSpec((1, A, D), lambda s: (s, 0, 0)),
            pl.BlockSpec((F, D * C), lambda s: (0, 0)),
            pl.BlockSpec((1, D * C), lambda s: (0, 0)),
        ],
        out_specs=pl.BlockSpec((1, A, C), lambda s: (s, 0, 0)),
        out_shape=jax.ShapeDtypeStruct((S, A, C), jnp.float32),
    )(atoms, bonds32, wp, br)
